# baseline (speedup 1.0000x reference)
import jax
import jax.numpy as jnp
from jax import lax
from jax.experimental import pallas as pl
from jax.experimental.pallas import tpu as pltpu
from jax.experimental.pallas import tpu_sc as plsc

NC = 2
NS = 16
NW = NC * NS

B_TOK, L_TOK = 4096, 200
D = 64
B = B_TOK * L_TOK
B_PER_W = B // NW
CHUNK = 128
N_CHUNKS = B_PER_W // CHUNK


def _body(tok_hbm, table_hbm, out_hbm, idx_v, rows_v, sem):
    c = lax.axis_index("c")
    s = lax.axis_index("s")
    wid = s * NC + c
    pltpu.sync_copy(tok_hbm.at[wid], idx_v)
    base = wid * B_PER_W

    def chunk(j, carry):
        pltpu.async_copy(table_hbm.at[idx_v.at[j]], rows_v, sem).wait()
        pltpu.sync_copy(rows_v, out_hbm.at[pl.ds(base + j * CHUNK, CHUNK)])
        return carry

    lax.fori_loop(0, N_CHUNKS, chunk, 0)


@jax.jit
def _embed(token_r, weight):
    mesh = plsc.VectorSubcoreMesh(core_axis_name="c", subcore_axis_name="s")
    return pl.kernel(
        _body,
        out_type=jax.ShapeDtypeStruct((B, D), jnp.float32),
        mesh=mesh,
        scratch_types=[
            pltpu.VMEM((N_CHUNKS, CHUNK), jnp.int32),
            pltpu.VMEM((CHUNK, D), jnp.float32),
            pltpu.SemaphoreType.DMA,
        ],
        compiler_params=pltpu.CompilerParams(use_tc_tiling_on_sc=False),
    )(token_r, weight)


def kernel(token_ids, weight):
    token_r = token_ids.reshape(NW, N_CHUNKS, CHUNK)
    out = _embed(token_r, weight)
    return out.reshape(B_TOK, L_TOK, D)


# speedup vs baseline: 1.1134x; 1.1134x over previous
import jax
import jax.numpy as jnp
from jax import lax
from jax.experimental import pallas as pl
from jax.experimental.pallas import tpu as pltpu
from jax.experimental.pallas import tpu_sc as plsc

NC = 2
NS = 16
NW = NC * NS

B_TOK, L_TOK = 4096, 200
D = 64
B = B_TOK * L_TOK
B_PER_W = B // NW
CHUNK = 128
N_CHUNKS = B_PER_W // CHUNK


NBUF = 4


def _body(tok_hbm, table_hbm, out_hbm, idx_v, rows_v, *sems):
    gsem = sems[:NBUF]
    ssem = sems[NBUF:]
    c = lax.axis_index("c")
    s = lax.axis_index("s")
    wid = s * NC + c
    pltpu.sync_copy(tok_hbm.at[wid], idx_v)
    base = wid * B_PER_W

    def gather_start(j, b):
        pltpu.async_copy(table_hbm.at[idx_v.at[j]], rows_v.at[b], gsem[b])

    for b in range(NBUF):
        gather_start(b, b)

    def outer(it, carry):
        g = it * NBUF
        for b in range(NBUF):
            j = g + b
            pltpu.make_async_copy(
                table_hbm.at[idx_v.at[j]], rows_v.at[b], gsem[b]
            ).wait()
            dst = out_hbm.at[pl.ds(base + j * CHUNK, CHUNK)]
            pltpu.async_copy(rows_v.at[b], dst, ssem[b])

            @pl.when(j + NBUF < N_CHUNKS)
            def _():
                pltpu.make_async_copy(rows_v.at[b], dst, ssem[b]).wait()
                gather_start(j + NBUF, b)

        return carry

    lax.fori_loop(0, N_CHUNKS // NBUF, outer, 0)
    for b in range(NBUF):
        j = N_CHUNKS - NBUF + b
        dst = out_hbm.at[pl.ds(base + j * CHUNK, CHUNK)]
        pltpu.make_async_copy(rows_v.at[b], dst, ssem[b]).wait()


@jax.jit
def _embed(token_r, weight):
    mesh = plsc.VectorSubcoreMesh(core_axis_name="c", subcore_axis_name="s")
    return pl.kernel(
        _body,
        out_type=jax.ShapeDtypeStruct((B, D), jnp.float32),
        mesh=mesh,
        scratch_types=[
            pltpu.VMEM((N_CHUNKS, CHUNK), jnp.int32),
            pltpu.VMEM((NBUF, CHUNK, D), jnp.float32),
        ] + [pltpu.SemaphoreType.DMA] * (2 * NBUF),
        compiler_params=pltpu.CompilerParams(use_tc_tiling_on_sc=False),
    )(token_r, weight)


def kernel(token_ids, weight):
    token_r = token_ids.reshape(NW, N_CHUNKS, CHUNK)
    out = _embed(token_r, weight)
    return out.reshape(B_TOK, L_TOK, D)


# speedup vs baseline: 1.1169x; 1.0031x over previous
import jax
import jax.numpy as jnp
from jax import lax
from jax.experimental import pallas as pl
from jax.experimental.pallas import tpu as pltpu
from jax.experimental.pallas import tpu_sc as plsc

NC = 2
NS = 16
NW = NC * NS

B_TOK, L_TOK = 4096, 200
D = 64
B = B_TOK * L_TOK
B_PER_W = B // NW
CHUNK = 128
N_CHUNKS = B_PER_W // CHUNK


NBUF = 8


def _body(tok_hbm, table_hbm, out_hbm, idx_v, rows_v, *sems):
    gsem = sems[:NBUF]
    ssem = sems[NBUF:]
    c = lax.axis_index("c")
    s = lax.axis_index("s")
    wid = s * NC + c
    pltpu.sync_copy(tok_hbm.at[wid], idx_v)
    base = wid * B_PER_W

    def gather_start(j, b):
        pltpu.async_copy(table_hbm.at[idx_v.at[j]], rows_v.at[b], gsem[b])

    for b in range(NBUF):
        gather_start(b, b)

    def outer(it, carry):
        g = it * NBUF
        for b in range(NBUF):
            j = g + b
            pltpu.make_async_copy(
                table_hbm.at[idx_v.at[j]], rows_v.at[b], gsem[b]
            ).wait()
            dst = out_hbm.at[pl.ds(base + j * CHUNK, CHUNK)]
            pltpu.async_copy(rows_v.at[b], dst, ssem[b])

            @pl.when(j + NBUF < N_CHUNKS)
            def _():
                pltpu.make_async_copy(rows_v.at[b], dst, ssem[b]).wait()
                gather_start(j + NBUF, b)

        return carry

    lax.fori_loop(0, N_CHUNKS // NBUF, outer, 0)
    for b in range(NBUF):
        j = N_CHUNKS - NBUF + b
        dst = out_hbm.at[pl.ds(base + j * CHUNK, CHUNK)]
        pltpu.make_async_copy(rows_v.at[b], dst, ssem[b]).wait()


@jax.jit
def _embed(token_r, weight):
    mesh = plsc.VectorSubcoreMesh(core_axis_name="c", subcore_axis_name="s")
    return pl.kernel(
        _body,
        out_type=jax.ShapeDtypeStruct((B, D), jnp.float32),
        mesh=mesh,
        scratch_types=[
            pltpu.VMEM((N_CHUNKS, CHUNK), jnp.int32),
            pltpu.VMEM((NBUF, CHUNK, D), jnp.float32),
        ] + [pltpu.SemaphoreType.DMA] * (2 * NBUF),
        compiler_params=pltpu.CompilerParams(use_tc_tiling_on_sc=False),
    )(token_r, weight)


def kernel(token_ids, weight):
    token_r = token_ids.reshape(NW, N_CHUNKS, CHUNK)
    out = _embed(token_r, weight)
    return out.reshape(B_TOK, L_TOK, D)
